# Initial kernel scaffold; baseline (speedup 1.0000x reference)
import jax
import jax.numpy as jnp
from jax.experimental import pallas as pl


def kernel(x, edge_index):
    raise NotImplementedError("write your pallas kernel here")


# speedup vs baseline: 8.8505x; 8.8505x over previous
import functools

import jax
import jax.numpy as jnp
from jax import lax
from jax.experimental import pallas as pl
from jax.experimental.pallas import tpu as pltpu
from jax.experimental.pallas import tpu_sc as plsc

N_NODES = 10000
N_EDGES = 320000
D_FEAT = 128

N_PAD = 10240
B_EDGES = 125
NBLK = N_EDGES // B_EDGES
NUM_CORES = 2
NUM_SUBCORES = 16
NUM_TILES = NUM_CORES * NUM_SUBCORES
BLKS_PER_TILE = NBLK // NUM_TILES
ROWS_PER_TILE = N_PAD // NUM_SUBCORES
ZROWS = 128


def _sc_gather_scatter(x, src2, dst2):
    mesh = plsc.VectorSubcoreMesh(core_axis_name="c", subcore_axis_name="s")

    @functools.partial(
        pl.kernel,
        out_type=jax.ShapeDtypeStruct((NUM_CORES, N_PAD, D_FEAT), jnp.float32),
        mesh=mesh,
        scratch_types=[
            pltpu.VMEM((BLKS_PER_TILE, B_EDGES), jnp.int32),
            pltpu.VMEM((BLKS_PER_TILE, B_EDGES), jnp.int32),
            pltpu.VMEM((ZROWS, D_FEAT), jnp.float32),
            pltpu.VMEM_SHARED((N_PAD, D_FEAT), jnp.float32),
        ],
    )
    def k(x_hbm, src_hbm, dst_hbm, out_hbm, src_v, dst_v, rows_v, acc):
        cid = lax.axis_index("c")
        sid = lax.axis_index("s")
        wid = cid * NUM_SUBCORES + sid

        zero = jnp.zeros((16,), jnp.float32)

        @pl.loop(0, ZROWS)
        def _(r):
            @pl.loop(0, D_FEAT // 16)
            def _(c):
                rows_v.at[r, pl.ds(c * 16, 16)][...] = zero

        @pl.loop(0, ROWS_PER_TILE // ZROWS)
        def _(z):
            pltpu.sync_copy(
                rows_v, acc.at[pl.ds(sid * ROWS_PER_TILE + z * ZROWS, ZROWS)])

        pltpu.sync_copy(src_hbm.at[pl.ds(wid * BLKS_PER_TILE, BLKS_PER_TILE)],
                        src_v)
        pltpu.sync_copy(dst_hbm.at[pl.ds(wid * BLKS_PER_TILE, BLKS_PER_TILE)],
                        dst_v)

        plsc.subcore_barrier()

        @pl.loop(0, BLKS_PER_TILE)
        def _(i):
            rows = rows_v.at[pl.ds(0, B_EDGES)]
            pltpu.sync_copy(x_hbm.at[src_v.at[i]], rows)
            pltpu.sync_copy(rows, acc.at[dst_v.at[i]], add=True)

        plsc.subcore_barrier()

        pltpu.sync_copy(
            acc.at[pl.ds(sid * ROWS_PER_TILE, ROWS_PER_TILE)],
            out_hbm.at[cid, pl.ds(sid * ROWS_PER_TILE, ROWS_PER_TILE)])

    return k(x, src2, dst2)


def _tc_combine(partial):
    def body(p_ref, o_ref):
        o_ref[...] = p_ref[0] + p_ref[1]

    nb = 10
    return pl.pallas_call(
        body,
        out_shape=jax.ShapeDtypeStruct((N_NODES, D_FEAT), jnp.float32),
        grid=(nb,),
        in_specs=[pl.BlockSpec((NUM_CORES, N_NODES // nb, D_FEAT),
                               lambda i: (0, i, 0))],
        out_specs=pl.BlockSpec((N_NODES // nb, D_FEAT), lambda i: (i, 0)),
    )(partial)


def kernel(x, edge_index):
    src2 = edge_index[0].reshape(NBLK, B_EDGES)
    dst2 = edge_index[1].reshape(NBLK, B_EDGES)
    partial = _sc_gather_scatter(x, src2, dst2)
    return _tc_combine(partial)
